# Initial kernel scaffold; baseline (speedup 1.0000x reference)
import jax
import jax.numpy as jnp
from jax.experimental import pallas as pl


def kernel(box_labels, ent_emb, rel_emb, norm_vec):
    raise NotImplementedError("write your pallas kernel here")


# speedup vs baseline: 6.3602x; 6.3602x over previous
import functools

import jax
import jax.numpy as jnp
from jax.experimental import pallas as pl
from jax.experimental.pallas import tpu as pltpu

N_H = 8
N = 64
NUM_CLS = 117
NUM_OBJ = 80
HUMAN_IDX = 49
DIM = 300
NUM_KEEP = N_H * N - N_H
PAIRS_PER_STEP = 8
NUM_STEPS = NUM_KEEP // PAIRS_PER_STEP


def _core_kernel(labels_ref, ent_ref, rel_ref, nv_ref,
                 out_h, out_r, out_w, out_t, out_s,
                 tn_s, hp_s, rn_s, wn_s, sc_s):
    step = pl.program_id(0)

    @pl.when(step == 0)
    def _compute_core():
        row = jax.lax.broadcasted_iota(jnp.int32, (N, 1), 0)
        labels = jnp.where(row < N_H, HUMAN_IDX, labels_ref[:])
        onehot = (labels == jax.lax.broadcasted_iota(
            jnp.int32, (N, NUM_OBJ), 1)).astype(jnp.float32)
        t_raw = jnp.dot(onehot, ent_ref[:], preferred_element_type=jnp.float32)

        def l2norm(x):
            return x / jnp.maximum(
                jnp.sqrt(jnp.sum(x * x, axis=1, keepdims=True)), 1e-12)

        t_n = l2norm(t_raw)
        h_n = l2norm(ent_ref[HUMAN_IDX:HUMAN_IDX + 1, :])
        r_n = l2norm(rel_ref[:])
        w_n = l2norm(nv_ref[:])

        h_dot = jnp.sum(h_n * w_n, axis=1, keepdims=True)
        h_p = h_n - h_dot * w_n

        tn_s[:] = t_n
        hp_s[:] = h_p
        rn_s[:] = r_n
        wn_s[:] = w_n

        a = h_p + r_n
        ones = jnp.ones((1, DIM), jnp.float32)
        aa_row = jax.lax.dot_general(
            ones, a * a, (((1,), (1,)), ((), ())),
            preferred_element_type=jnp.float32)
        aw_row = jax.lax.dot_general(
            ones, a * w_n, (((1,), (1,)), ((), ())),
            preferred_element_type=jnp.float32)
        t_dot = jax.lax.dot_general(
            t_n, w_n, (((1,), (1,)), ((), ())),
            preferred_element_type=jnp.float32)
        a_t = jax.lax.dot_general(
            t_n, a, (((1,), (1,)), ((), ())),
            preferred_element_type=jnp.float32)
        tt = jnp.sum(t_n * t_n, axis=1, keepdims=True)
        sq = aa_row + tt - t_dot * t_dot - 2.0 * a_t + 2.0 * t_dot * aw_row
        sc_s[:] = jnp.sqrt(jnp.maximum(sq, 0.0))

    hp = hp_s[:]
    rn = rn_s[:]
    wn = wn_s[:]
    out_h[:] = jnp.broadcast_to(hp[None], (PAIRS_PER_STEP, NUM_CLS, DIM))
    out_r[:] = jnp.broadcast_to(rn[None], (PAIRS_PER_STEP, NUM_CLS, DIM))
    out_w[:] = jnp.broadcast_to(wn[None], (PAIRS_PER_STEP, NUM_CLS, DIM))

    i_col = jax.lax.broadcasted_iota(jnp.int32, (PAIRS_PER_STEP, 1), 0)
    k = step * PAIRS_PER_STEP + i_col
    x = k // (N - 1)
    j = k - x * (N - 1)
    y = jnp.where(j >= x, j + 1, j)
    sel = (y == jax.lax.broadcasted_iota(
        jnp.int32, (PAIRS_PER_STEP, N), 1)).astype(jnp.float32)

    out_s[:] = jnp.dot(sel, sc_s[:], preferred_element_type=jnp.float32)
    tn_step = jnp.dot(sel, tn_s[:], preferred_element_type=jnp.float32)

    for i in range(PAIRS_PER_STEP):
        tn = tn_step[i:i + 1, :]
        td = jnp.sum(tn * wn, axis=1, keepdims=True)
        out_t[i] = tn - td * wn


@jax.jit
def kernel(box_labels, ent_emb, rel_emb, norm_vec):
    labels2d = box_labels.reshape(N, 1)
    big = jax.ShapeDtypeStruct((NUM_KEEP, NUM_CLS, DIM), jnp.float32)
    outs = pl.pallas_call(
        _core_kernel,
        grid=(NUM_STEPS,),
        in_specs=[
            pl.BlockSpec((N, 1), lambda s: (0, 0)),
            pl.BlockSpec((NUM_OBJ, DIM), lambda s: (0, 0)),
            pl.BlockSpec((NUM_CLS, DIM), lambda s: (0, 0)),
            pl.BlockSpec((NUM_CLS, DIM), lambda s: (0, 0)),
        ],
        out_specs=[
            pl.BlockSpec((PAIRS_PER_STEP, NUM_CLS, DIM), lambda s: (s, 0, 0)),
            pl.BlockSpec((PAIRS_PER_STEP, NUM_CLS, DIM), lambda s: (s, 0, 0)),
            pl.BlockSpec((PAIRS_PER_STEP, NUM_CLS, DIM), lambda s: (s, 0, 0)),
            pl.BlockSpec((PAIRS_PER_STEP, NUM_CLS, DIM), lambda s: (s, 0, 0)),
            pl.BlockSpec((PAIRS_PER_STEP, NUM_CLS), lambda s: (s, 0)),
        ],
        out_shape=[big, big, big, big,
                   jax.ShapeDtypeStruct((NUM_KEEP, NUM_CLS), jnp.float32)],
        scratch_shapes=[
            pltpu.VMEM((N, DIM), jnp.float32),
            pltpu.VMEM((NUM_CLS, DIM), jnp.float32),
            pltpu.VMEM((NUM_CLS, DIM), jnp.float32),
            pltpu.VMEM((NUM_CLS, DIM), jnp.float32),
            pltpu.VMEM((N, NUM_CLS), jnp.float32),
        ],
    )(labels2d, ent_emb, rel_emb, norm_vec)
    h_keep, r_keep, w_keep, t_keep, scores_keep = outs
    return (h_keep, r_keep, w_keep, t_keep, scores_keep)
